# Initial kernel scaffold; baseline (speedup 1.0000x reference)
import jax
import jax.numpy as jnp
from jax.experimental import pallas as pl


def kernel(user_indices, item_indices, edge_index, embedding):
    raise NotImplementedError("write your pallas kernel here")


# speedup vs baseline: 2.1649x; 2.1649x over previous
import jax
import jax.numpy as jnp
from jax import lax
from jax.experimental import pallas as pl
from jax.experimental.pallas import tpu as pltpu
from jax.experimental.pallas import tpu_sc as plsc

N_USERS = 25000
N_NODES = 50000
N_EDGES = 800000
DIM = 64
NQ = 4
HDIM = DIM // NQ
BATCH = 4096

NC = 2
NS = 16
LANES = 16

HALF = N_NODES // NC
ACC_ROWS = HALF + 8

EPT = N_EDGES // NS
C = 80
NCHUNK = EPT // C
ROW_SPAN = 1568

_mesh = plsc.VectorSubcoreMesh(core_axis_name="c", subcore_axis_name="s")
_params = pltpu.CompilerParams(use_tc_tiling_on_sc=False)
_params_ro = pltpu.CompilerParams(use_tc_tiling_on_sc=False,
                                  needs_layout_passes=False)


def _propagate_body(src_hbm, dst_hbm, cur_hbm, zero_hbm, out_hbm,
                    sidx, dall, didx0, didx1, rows0, rows1, zbuf,
                    acc, sem0, sem1):
    sc = lax.axis_index("c")
    s = lax.axis_index("s")
    sc_off = sc * HALF
    tile_edge0 = s * EPT

    pltpu.sync_copy(src_hbm.at[pl.ds(tile_edge0, EPT)], sidx)
    pltpu.sync_copy(dst_hbm.at[pl.ds(tile_edge0, EPT)], dall)
    pltpu.sync_copy(zero_hbm, zbuf)

    def transform(i, didx):
        base = i * C
        for k in range(C // LANES):
            v = dall[pl.ds(base + k * LANES, LANES)]
            lo = v - sc_off
            ok = (lo >= 0) & (lo < HALF)
            didx[pl.ds(k * LANES, LANES)] = jnp.where(ok, lo, HALF + k)

    def issue(i, rows_ref, sem):
        pltpu.async_copy(cur_hbm.at[sidx.at[pl.ds(i * C, C)]], rows_ref, sem)

    def wait_gather(rows_ref, sem):
        pltpu.make_async_copy(cur_hbm.at[pl.ds(0, C), :], rows_ref, sem).wait()

    issue(0, rows0, sem0)
    issue(1, rows1, sem1)

    for j in range(-(-ROW_SPAN // C)):
        b = jnp.minimum(s * ROW_SPAN + j * C, ACC_ROWS - C)
        pltpu.sync_copy(zbuf, acc.at[pl.ds(b, C), :])
    plsc.subcore_barrier()

    def pair_body(p, carry):
        a = 2 * p
        wait_gather(rows0, sem0)
        transform(a, didx0)
        pltpu.sync_copy(rows0, acc.at[didx0], add=True)
        issue(a + 2, rows0, sem0)
        wait_gather(rows1, sem1)
        transform(a + 1, didx1)
        pltpu.sync_copy(rows1, acc.at[didx1], add=True)

        @pl.when(p < NCHUNK // 2 - 1)
        def _():
            issue(a + 3, rows1, sem1)
        return carry

    lax.fori_loop(0, NCHUNK // 2, pair_body, 0)
    wait_gather(rows0, sem0)
    transform(NCHUNK - 1, didx0)
    pltpu.sync_copy(rows0, acc.at[didx0], add=True)

    plsc.subcore_barrier()

    for j in range(-(-ROW_SPAN // C)):
        b = jnp.minimum(s * ROW_SPAN + j * C, HALF - C)
        pltpu.sync_copy(acc.at[pl.ds(b, C), :], rows0)
        pltpu.sync_copy(rows0, out_hbm.at[pl.ds(sc_off + b, C), :])


_propagate = pl.kernel(
    _propagate_body,
    out_type=jax.ShapeDtypeStruct((N_NODES, HDIM), jnp.float32),
    mesh=_mesh,
    compiler_params=_params,
    scratch_types=[
        pltpu.VMEM((EPT,), jnp.int32),
        pltpu.VMEM((EPT,), jnp.int32),
        pltpu.VMEM((C,), jnp.int32),
        pltpu.VMEM((C,), jnp.int32),
        pltpu.VMEM((C, HDIM), jnp.float32),
        pltpu.VMEM((C, HDIM), jnp.float32),
        pltpu.VMEM((C, HDIM), jnp.float32),
        pltpu.VMEM_SHARED((ACC_ROWS, HDIM), jnp.float32),
        pltpu.SemaphoreType.DMA,
        pltpu.SemaphoreType.DMA,
    ],
)


BPT = BATCH // (NC * NS)


def _readout_body(*refs):
    (uidx_hbm, iidx_hbm), tables, (out_hbm,) = refs[:2], refs[2:18], refs[18:19]
    uidx, iidx, uacc, iacc, tmp, outv, sem = refs[19:]
    sc = lax.axis_index("c")
    s = lax.axis_index("s")
    w = sc * NS + s
    base = w * BPT

    pltpu.sync_copy(uidx_hbm.at[pl.ds(base, BPT)], uidx)
    pltpu.sync_copy(iidx_hbm.at[pl.ds(base, BPT)], iidx)
    for k in range(BPT // LANES):
        iidx[pl.ds(k * LANES, LANES)] = (
            iidx[pl.ds(k * LANES, LANES)] + N_USERS)

    def accumulate(idx_ref, acc_ref):
        for q in range(NQ):
            qsl = pl.ds(q * HDIM, HDIM)
            for l in range(4):
                pltpu.async_copy(tables[q * 4 + l].at[idx_ref], tmp, sem).wait()

                def add_row(r, carry):
                    if l == 0:
                        acc_ref[r, qsl] = tmp[r, pl.ds(0, HDIM)]
                    else:
                        acc_ref[r, qsl] = acc_ref[r, qsl] + tmp[r, pl.ds(0, HDIM)]
                    return carry
                lax.fori_loop(0, BPT, add_row, 0)

    accumulate(uidx, uacc)
    accumulate(iidx, iacc)

    def mul_row(r, carry):
        for cch in range(DIM // LANES):
            sl = pl.ds(cch * LANES, LANES)
            uacc[r, sl] = uacc[r, sl] * iacc[r, sl]
        return carry
    lax.fori_loop(0, BPT, mul_row, 0)

    row_iota = lax.iota(jnp.int32, LANES)
    for g in range(BPT // LANES):
        rows = row_iota + (g * LANES)

        def col_body(col, acc16):
            cols = jnp.full((LANES,), 0, jnp.int32) + col
            return acc16 + plsc.load_gather(uacc, [rows, cols])
        acc16 = lax.fori_loop(0, DIM, col_body,
                              jnp.zeros((LANES,), jnp.float32))
        outv[pl.ds(g * LANES, LANES)] = acc16 * 0.0625

    pltpu.sync_copy(outv, out_hbm.at[pl.ds(base, BPT)])


_readout = pl.kernel(
    _readout_body,
    out_type=jax.ShapeDtypeStruct((BATCH,), jnp.float32),
    mesh=_mesh,
    compiler_params=_params_ro,
    scratch_types=[
        pltpu.VMEM((BPT,), jnp.int32),
        pltpu.VMEM((BPT,), jnp.int32),
        pltpu.VMEM((BPT, DIM), jnp.float32),
        pltpu.VMEM((BPT, DIM), jnp.float32),
        pltpu.VMEM((BPT, HDIM), jnp.float32),
        pltpu.VMEM((BPT,), jnp.float32),
        pltpu.SemaphoreType.DMA,
    ],
)


def kernel(user_indices, item_indices, edge_index, embedding):
    src = edge_index[0]
    dst = edge_index[1]
    emb_q = jnp.transpose(embedding.reshape(N_NODES, NQ, HDIM), (1, 0, 2))
    zeros_blk = jnp.zeros((C, HDIM), jnp.float32)

    def layer_step(cur, _):
        nxt = _propagate(src, dst, cur, zeros_blk)
        return nxt, nxt

    def quarter_step(carry, embq):
        _, layers_q = lax.scan(layer_step, embq, None, length=3)
        return carry, layers_q

    _, layers = lax.scan(quarter_step, 0, emb_q)

    tables = []
    for q in range(NQ):
        tables.append(emb_q[q])
        for l in range(3):
            tables.append(layers[q, l])
    dots = _readout(user_indices, item_indices, *tables)
    return dots.reshape(BATCH, 1)


# speedup vs baseline: 3.5812x; 1.6543x over previous
import jax
import jax.numpy as jnp
from jax import lax
from jax.experimental import pallas as pl
from jax.experimental.pallas import tpu as pltpu
from jax.experimental.pallas import tpu_sc as plsc

N_USERS = 25000
N_NODES = 50000
N_EDGES = 800000
DIM = 64
NQ = 4
HDIM = 32
QDIM = 16
BATCH = 4096

NC = 2
NS = 16
LANES = 16

HALF = N_NODES // NC
ACC_ROWS = HALF + 8

EPT = N_EDGES // NS
C = 80
NCHUNK = EPT // C
NSEC = 5
SECC = NCHUNK // NSEC
SECE = SECC * C
ROW_SPAN = 1568

_mesh = plsc.VectorSubcoreMesh(core_axis_name="c", subcore_axis_name="s")
_params = pltpu.CompilerParams(use_tc_tiling_on_sc=False)
_params_ro = pltpu.CompilerParams(use_tc_tiling_on_sc=False,
                                  needs_layout_passes=False)


def _propagate_body(src_hbm, dst_hbm, cur_hbm, zero_hbm, out_hbm,
                    sidx, dall, didx0, didx1, rows0, rows1, zbuf,
                    acc, sem0, sem1):
    sc = lax.axis_index("c")
    s = lax.axis_index("s")
    sc_off = sc * HALF
    tile_edge0 = s * EPT

    pltpu.sync_copy(zero_hbm, zbuf)

    def transform(i, didx):
        base = i * C
        for k in range(C // LANES):
            v = dall[pl.ds(base + k * LANES, LANES)]
            lo = v - sc_off
            ok = (lo >= 0) & (lo < HALF)
            didx[pl.ds(k * LANES, LANES)] = jnp.where(ok, lo, HALF + k)

    def issue(i, rows_ref, sem):
        pltpu.async_copy(cur_hbm.at[sidx.at[pl.ds(i * C, C)]], rows_ref, sem)

    def wait_gather(rows_ref, sem):
        pltpu.make_async_copy(cur_hbm.at[pl.ds(0, C), :], rows_ref, sem).wait()

    for j in range(-(-ROW_SPAN // C)):
        b = jnp.minimum(s * ROW_SPAN + j * C, ACC_ROWS - C)
        pltpu.sync_copy(zbuf, acc.at[pl.ds(b, C), :])
    plsc.subcore_barrier()

    for sec in range(NSEC):
        sbase = tile_edge0 + sec * SECE
        pltpu.sync_copy(src_hbm.at[pl.ds(sbase, SECE)], sidx)
        pltpu.sync_copy(dst_hbm.at[pl.ds(sbase, SECE)], dall)
        issue(0, rows0, sem0)
        issue(1, rows1, sem1)

        def pair_body(p, carry):
            a = 2 * p
            wait_gather(rows0, sem0)
            transform(a, didx0)
            pltpu.sync_copy(rows0, acc.at[didx0], add=True)
            issue(a + 2, rows0, sem0)
            wait_gather(rows1, sem1)
            transform(a + 1, didx1)
            pltpu.sync_copy(rows1, acc.at[didx1], add=True)

            @pl.when(p < SECC // 2 - 1)
            def _():
                issue(a + 3, rows1, sem1)
            return carry

        lax.fori_loop(0, SECC // 2, pair_body, 0)
        wait_gather(rows0, sem0)
        transform(SECC - 1, didx0)
        pltpu.sync_copy(rows0, acc.at[didx0], add=True)

    plsc.subcore_barrier()

    for j in range(-(-ROW_SPAN // C)):
        b = jnp.minimum(s * ROW_SPAN + j * C, HALF - C)
        pltpu.sync_copy(acc.at[pl.ds(b, C), :], rows0)
        pltpu.sync_copy(rows0, out_hbm.at[pl.ds(sc_off + b, C), :])


_propagate = pl.kernel(
    _propagate_body,
    out_type=jax.ShapeDtypeStruct((N_NODES, HDIM), jnp.float32),
    mesh=_mesh,
    compiler_params=_params,
    scratch_types=[
        pltpu.VMEM((SECE,), jnp.int32),
        pltpu.VMEM((SECE,), jnp.int32),
        pltpu.VMEM((C,), jnp.int32),
        pltpu.VMEM((C,), jnp.int32),
        pltpu.VMEM((C, HDIM), jnp.float32),
        pltpu.VMEM((C, HDIM), jnp.float32),
        pltpu.VMEM((C, HDIM), jnp.float32),
        pltpu.VMEM_SHARED((ACC_ROWS, HDIM), jnp.float32),
        pltpu.SemaphoreType.DMA,
        pltpu.SemaphoreType.DMA,
    ],
)


BPT = BATCH // (NC * NS)


def _readout_body(*refs):
    (uidx_hbm, iidx_hbm), tables, (out_hbm,) = refs[:2], refs[2:18], refs[18:19]
    uidx, iidx, uacc, iacc, tmp, outv, sem = refs[19:]
    sc = lax.axis_index("c")
    s = lax.axis_index("s")
    w = sc * NS + s
    base = w * BPT

    pltpu.sync_copy(uidx_hbm.at[pl.ds(base, BPT)], uidx)
    pltpu.sync_copy(iidx_hbm.at[pl.ds(base, BPT)], iidx)
    for k in range(BPT // LANES):
        iidx[pl.ds(k * LANES, LANES)] = (
            iidx[pl.ds(k * LANES, LANES)] + N_USERS)

    def accumulate(idx_ref, acc_ref):
        for q in range(NQ):
            qsl = pl.ds(q * QDIM, QDIM)
            for l in range(4):
                pltpu.async_copy(tables[q * 4 + l].at[idx_ref], tmp, sem).wait()

                def add_row(r, carry):
                    if l == 0:
                        acc_ref[r, qsl] = tmp[r, pl.ds(0, QDIM)]
                    else:
                        acc_ref[r, qsl] = acc_ref[r, qsl] + tmp[r, pl.ds(0, QDIM)]
                    return carry
                lax.fori_loop(0, BPT, add_row, 0)

    accumulate(uidx, uacc)
    accumulate(iidx, iacc)

    def mul_row(r, carry):
        for cch in range(DIM // LANES):
            sl = pl.ds(cch * LANES, LANES)
            uacc[r, sl] = uacc[r, sl] * iacc[r, sl]
        return carry
    lax.fori_loop(0, BPT, mul_row, 0)

    row_iota = lax.iota(jnp.int32, LANES)
    for g in range(BPT // LANES):
        rows = row_iota + (g * LANES)

        def col_body(col, acc16):
            cols = jnp.full((LANES,), 0, jnp.int32) + col
            return acc16 + plsc.load_gather(uacc, [rows, cols])
        acc16 = lax.fori_loop(0, DIM, col_body,
                              jnp.zeros((LANES,), jnp.float32))
        outv[pl.ds(g * LANES, LANES)] = acc16 * 0.0625

    pltpu.sync_copy(outv, out_hbm.at[pl.ds(base, BPT)])


_readout = pl.kernel(
    _readout_body,
    out_type=jax.ShapeDtypeStruct((BATCH,), jnp.float32),
    mesh=_mesh,
    compiler_params=_params_ro,
    scratch_types=[
        pltpu.VMEM((BPT,), jnp.int32),
        pltpu.VMEM((BPT,), jnp.int32),
        pltpu.VMEM((BPT, DIM), jnp.float32),
        pltpu.VMEM((BPT, DIM), jnp.float32),
        pltpu.VMEM((BPT, QDIM), jnp.float32),
        pltpu.VMEM((BPT,), jnp.float32),
        pltpu.SemaphoreType.DMA,
    ],
)


def kernel(user_indices, item_indices, edge_index, embedding):
    src = edge_index[0]
    dst = edge_index[1]
    emb_p = jnp.transpose(embedding.reshape(N_NODES, 2, HDIM), (1, 0, 2))
    zeros_blk = jnp.zeros((C, HDIM), jnp.float32)

    def layer_step(cur, _):
        nxt = _propagate(src, dst, cur, zeros_blk)
        return nxt, nxt

    def pair_step(carry, embp):
        _, layers_p = lax.scan(layer_step, embp, None, length=3)
        return carry, layers_p

    _, layers = lax.scan(pair_step, 0, emb_p)

    tables = []
    for q in range(NQ):
        p, h = q // 2, q % 2
        tables.append(emb_p[p, :, h * QDIM:(h + 1) * QDIM])
        for l in range(3):
            tables.append(layers[p, l, :, h * QDIM:(h + 1) * QDIM])
    dots = _readout(user_indices, item_indices, *tables)
    return dots.reshape(BATCH, 1)


# speedup vs baseline: 5.0773x; 1.4177x over previous
import jax
import jax.numpy as jnp
from jax import lax
from jax.experimental import pallas as pl
from jax.experimental.pallas import tpu as pltpu
from jax.experimental.pallas import tpu_sc as plsc

N_USERS = 25000
N_NODES = 50000
N_EDGES = 800000
DIM = 64
NQ = 4
HDIM = 64
QDIM = 16
BATCH = 4096

NC = 2
NS = 16
LANES = 16

HALF = N_NODES // NC
ACC_ROWS = HALF + 8

EPT = N_EDGES // NS
C = 80
NCHUNK = EPT // C
NSEC = 25
SECC = NCHUNK // NSEC
SECE = SECC * C
ROW_SPAN = 1568

_mesh = plsc.VectorSubcoreMesh(core_axis_name="c", subcore_axis_name="s")
_params = pltpu.CompilerParams(use_tc_tiling_on_sc=False)
_params_ro = pltpu.CompilerParams(use_tc_tiling_on_sc=False,
                                  needs_layout_passes=False)


def _propagate_body(src_hbm, dst_hbm, cur_hbm, zero_hbm, out_hbm,
                    sidx, dall, didx0, didx1, rows0, rows1, zbuf,
                    acc, sem0, sem1):
    sc = lax.axis_index("c")
    s = lax.axis_index("s")
    sc_off = sc * HALF
    tile_edge0 = s * EPT

    pltpu.sync_copy(zero_hbm, zbuf)

    def transform(i, didx):
        base = i * C
        for k in range(C // LANES):
            v = dall[pl.ds(base + k * LANES, LANES)]
            lo = v - sc_off
            ok = (lo >= 0) & (lo < HALF)
            didx[pl.ds(k * LANES, LANES)] = jnp.where(ok, lo, HALF + k)

    def issue(i, rows_ref, sem):
        pltpu.async_copy(cur_hbm.at[sidx.at[pl.ds(i * C, C)]], rows_ref, sem)

    def wait_gather(rows_ref, sem):
        pltpu.make_async_copy(cur_hbm.at[pl.ds(0, C), :], rows_ref, sem).wait()

    for j in range(-(-ROW_SPAN // C)):
        b = jnp.minimum(s * ROW_SPAN + j * C, ACC_ROWS - C)
        pltpu.sync_copy(zbuf, acc.at[pl.ds(b, C), :])
    plsc.subcore_barrier()

    for sec in range(NSEC):
        sbase = tile_edge0 + sec * SECE
        pltpu.sync_copy(src_hbm.at[pl.ds(sbase, SECE)], sidx)
        pltpu.sync_copy(dst_hbm.at[pl.ds(sbase, SECE)], dall)
        issue(0, rows0, sem0)
        issue(1, rows1, sem1)

        def pair_body(p, carry):
            a = 2 * p
            wait_gather(rows0, sem0)
            transform(a, didx0)
            pltpu.sync_copy(rows0, acc.at[didx0], add=True)
            issue(a + 2, rows0, sem0)
            wait_gather(rows1, sem1)
            transform(a + 1, didx1)
            pltpu.sync_copy(rows1, acc.at[didx1], add=True)

            @pl.when(p < SECC // 2 - 1)
            def _():
                issue(a + 3, rows1, sem1)
            return carry

        lax.fori_loop(0, SECC // 2, pair_body, 0)
        wait_gather(rows0, sem0)
        transform(SECC - 1, didx0)
        pltpu.sync_copy(rows0, acc.at[didx0], add=True)

    plsc.subcore_barrier()

    for j in range(-(-ROW_SPAN // C)):
        b = jnp.minimum(s * ROW_SPAN + j * C, HALF - C)
        pltpu.sync_copy(acc.at[pl.ds(b, C), :], rows0)
        pltpu.sync_copy(rows0, out_hbm.at[pl.ds(sc_off + b, C), :])


_propagate = pl.kernel(
    _propagate_body,
    out_type=jax.ShapeDtypeStruct((N_NODES, HDIM), jnp.float32),
    mesh=_mesh,
    compiler_params=_params,
    scratch_types=[
        pltpu.VMEM((SECE,), jnp.int32),
        pltpu.VMEM((SECE,), jnp.int32),
        pltpu.VMEM((C,), jnp.int32),
        pltpu.VMEM((C,), jnp.int32),
        pltpu.VMEM((C, HDIM), jnp.float32),
        pltpu.VMEM((C, HDIM), jnp.float32),
        pltpu.VMEM((C, HDIM), jnp.float32),
        pltpu.VMEM_SHARED((ACC_ROWS, HDIM), jnp.float32),
        pltpu.SemaphoreType.DMA,
        pltpu.SemaphoreType.DMA,
    ],
)


BPT = BATCH // (NC * NS)


def _readout_body(*refs):
    (uidx_hbm, iidx_hbm), tables, (out_hbm,) = refs[:2], refs[2:18], refs[18:19]
    uidx, iidx, uacc, iacc, tmp, outv, sem = refs[19:]
    sc = lax.axis_index("c")
    s = lax.axis_index("s")
    w = sc * NS + s
    base = w * BPT

    pltpu.sync_copy(uidx_hbm.at[pl.ds(base, BPT)], uidx)
    pltpu.sync_copy(iidx_hbm.at[pl.ds(base, BPT)], iidx)
    for k in range(BPT // LANES):
        iidx[pl.ds(k * LANES, LANES)] = (
            iidx[pl.ds(k * LANES, LANES)] + N_USERS)

    def accumulate(idx_ref, acc_ref):
        for q in range(NQ):
            qsl = pl.ds(q * QDIM, QDIM)
            for l in range(4):
                pltpu.async_copy(tables[q * 4 + l].at[idx_ref], tmp, sem).wait()

                def add_row(r, carry):
                    if l == 0:
                        acc_ref[r, qsl] = tmp[r, pl.ds(0, QDIM)]
                    else:
                        acc_ref[r, qsl] = acc_ref[r, qsl] + tmp[r, pl.ds(0, QDIM)]
                    return carry
                lax.fori_loop(0, BPT, add_row, 0)

    accumulate(uidx, uacc)
    accumulate(iidx, iacc)

    def mul_row(r, carry):
        for cch in range(DIM // LANES):
            sl = pl.ds(cch * LANES, LANES)
            uacc[r, sl] = uacc[r, sl] * iacc[r, sl]
        return carry
    lax.fori_loop(0, BPT, mul_row, 0)

    row_iota = lax.iota(jnp.int32, LANES)
    for g in range(BPT // LANES):
        rows = row_iota + (g * LANES)

        def col_body(col, acc16):
            cols = jnp.full((LANES,), 0, jnp.int32) + col
            return acc16 + plsc.load_gather(uacc, [rows, cols])
        acc16 = lax.fori_loop(0, DIM, col_body,
                              jnp.zeros((LANES,), jnp.float32))
        outv[pl.ds(g * LANES, LANES)] = acc16 * 0.0625

    pltpu.sync_copy(outv, out_hbm.at[pl.ds(base, BPT)])


_readout = pl.kernel(
    _readout_body,
    out_type=jax.ShapeDtypeStruct((BATCH,), jnp.float32),
    mesh=_mesh,
    compiler_params=_params_ro,
    scratch_types=[
        pltpu.VMEM((BPT,), jnp.int32),
        pltpu.VMEM((BPT,), jnp.int32),
        pltpu.VMEM((BPT, DIM), jnp.float32),
        pltpu.VMEM((BPT, DIM), jnp.float32),
        pltpu.VMEM((BPT, QDIM), jnp.float32),
        pltpu.VMEM((BPT,), jnp.float32),
        pltpu.SemaphoreType.DMA,
    ],
)


def kernel(user_indices, item_indices, edge_index, embedding):
    src = edge_index[0]
    dst = edge_index[1]
    zeros_blk = jnp.zeros((C, HDIM), jnp.float32)

    def layer_step(cur, _):
        nxt = _propagate(src, dst, cur, zeros_blk)
        return nxt, nxt

    _, layers = lax.scan(layer_step, embedding, None, length=3)

    tables = []
    for q in range(NQ):
        tables.append(embedding[:, q * QDIM:(q + 1) * QDIM])
        for l in range(3):
            tables.append(layers[l, :, q * QDIM:(q + 1) * QDIM])
    dots = _readout(user_indices, item_indices, *tables)
    return dots.reshape(BATCH, 1)
